# baseline (speedup 1.0000x reference)
import jax
import jax.numpy as jnp
from jax import lax
from jax.experimental import pallas as pl
from jax.experimental.pallas import tpu as pltpu
from jax.experimental.pallas import tpu_sc as plsc

N = 10000
E = 160000
NC = 2
NS = 16
LANES = 128

EPT = E // NS
CH = 80
NCHUNK = EPT // CH
NBUF = 2
NP = 10112
RPT = NP // NS

NW = NC * NS
EPW = E // NW
CW = 128
CCH = 40
NCC = EPW // CCH

_MESH = plsc.VectorSubcoreMesh(core_axis_name="c", subcore_axis_name="s")


def _cnt_body(dst2w_hbm, zcnt, ones_hbm, cnt_hbm, cntacc, dstlall, onesbuf):
  c = lax.axis_index("c")
  s = lax.axis_index("s")
  w = s * NC + c

  pltpu.sync_copy(zcnt, cntacc.at[pl.ds(s * RPT, RPT)])
  pltpu.sync_copy(ones_hbm, onesbuf)
  pltpu.sync_copy(dst2w_hbm.at[w], dstlall)
  plsc.subcore_barrier()

  def chunk(k, _):
    pltpu.sync_copy(onesbuf, cntacc.at[dstlall.at[pl.ds(k * CCH, CCH)]],
                    add=True)
    return 0
  lax.fori_loop(0, NCC, chunk, 0)

  plsc.subcore_barrier()
  pltpu.sync_copy(cntacc.at[pl.ds(s * RPT, RPT)],
                  cnt_hbm.at[c, pl.ds(s * RPT, RPT)])


_sc_cnt = pl.kernel(
    _cnt_body,
    out_type=jax.ShapeDtypeStruct((NC, NP, CW), jnp.float32),
    mesh=_MESH,
    scratch_types=[
        pltpu.VMEM_SHARED((NP, CW), jnp.float32),
        pltpu.VMEM((EPW,), jnp.int32),
        pltpu.VMEM((CCH, CW), jnp.float32),
    ],
)


def _agg_body(u3_hbm, src2_hbm, dst2_hbm, zrow, agg_hbm, acc, srcall, dstall,
              b0, b1, s0, s1):
  c = lax.axis_index("c")
  s = lax.axis_index("s")
  bufs = (b0, b1)
  sems = (s0, s1)

  pltpu.sync_copy(zrow, acc.at[pl.ds(s * RPT, RPT)])
  pltpu.sync_copy(src2_hbm.at[s], srcall)
  pltpu.sync_copy(dst2_hbm.at[s], dstall)
  plsc.subcore_barrier()

  uc = u3_hbm.at[c]

  def fire(k, b):
    pltpu.async_copy(uc.at[srcall.at[pl.ds(k * CH, CH)]], bufs[b], sems[b])

  def finish(k, b):
    pltpu.make_async_copy(uc.at[srcall.at[pl.ds(0, CH)]],
                          bufs[b], sems[b]).wait()
    pltpu.sync_copy(bufs[b], acc.at[dstall.at[pl.ds(k * CH, CH)]],
                    add=True)

  fire(0, 0)
  fire(1, 1)

  def outer(j, _):
    for b in range(NBUF):
      k = j * NBUF + b
      finish(k, b)
      fire(k + NBUF, b)
    return 0
  lax.fori_loop(0, (NCHUNK - 3) // NBUF, outer, 0)

  finish(NCHUNK - 3, 0)
  fire(NCHUNK - 1, 0)
  finish(NCHUNK - 2, 1)
  finish(NCHUNK - 1, 0)

  plsc.subcore_barrier()
  pltpu.sync_copy(acc.at[pl.ds(s * RPT, RPT)],
                  agg_hbm.at[c, pl.ds(s * RPT, RPT)])


_sc_agg = pl.kernel(
    _agg_body,
    out_type=jax.ShapeDtypeStruct((NC, NP, LANES), jnp.float32),
    mesh=_MESH,
    scratch_types=[
        pltpu.VMEM_SHARED((NP, LANES), jnp.float32),
        pltpu.VMEM((EPT,), jnp.int32),
        pltpu.VMEM((EPT,), jnp.int32),
    ] + [pltpu.VMEM((CH, LANES), jnp.float32) for _ in range(NBUF)]
      + [pltpu.SemaphoreType.DMA for _ in range(NBUF)],
)


BM = 1000


def _prep_body(x_ref, c0_ref, c1_ref, u_ref):
  d = lax.rsqrt(c0_ref[...] + c1_ref[...] + 1.0)
  ux = d * x_ref[...]
  u_ref[0] = ux[:, :LANES]
  u_ref[1] = ux[:, LANES:]


def _tc_prep(x, c0, c1):
  return pl.pallas_call(
      _prep_body,
      grid=(N // BM,),
      in_specs=[
          pl.BlockSpec((BM, 256), lambda i: (i, 0)),
          pl.BlockSpec((BM, 1), lambda i: (i, 0)),
          pl.BlockSpec((BM, 1), lambda i: (i, 0)),
      ],
      out_specs=pl.BlockSpec((2, BM, LANES), lambda i: (0, i, 0)),
      out_shape=jax.ShapeDtypeStruct((2, N, LANES), jnp.float32),
  )(x, c0, c1)


def _layer1_body(a_ref, u_ref, c0_ref, c1_ref, w_ref, b_ref, h_ref):
  d = lax.rsqrt(c0_ref[...] + c1_ref[...] + 1.0)
  agg = jnp.concatenate([a_ref[0], a_ref[1]], axis=1)
  uu = jnp.concatenate([u_ref[0], u_ref[1]], axis=1)
  p = d * (agg + uu)
  h = jnp.dot(p, w_ref[...], preferred_element_type=jnp.float32) + b_ref[...]
  h_ref[...] = jnp.maximum(h, 0.0)


def _tc_layer1(aggp, u, c0, c1, W1, b1):
  return pl.pallas_call(
      _layer1_body,
      grid=(N // BM,),
      in_specs=[
          pl.BlockSpec((2, BM, LANES), lambda i: (0, i, 0)),
          pl.BlockSpec((2, BM, LANES), lambda i: (0, i, 0)),
          pl.BlockSpec((BM, 1), lambda i: (i, 0)),
          pl.BlockSpec((BM, 1), lambda i: (i, 0)),
          pl.BlockSpec((256, 512), lambda i: (0, 0)),
          pl.BlockSpec((1, 512), lambda i: (0, 0)),
      ],
      out_specs=pl.BlockSpec((BM, 512), lambda i: (i, 0)),
      out_shape=jax.ShapeDtypeStruct((N, 512), jnp.float32),
  )(aggp, u, c0, c1, W1, b1)


def _u2_body(h_ref, c0_ref, c1_ref, w_ref, u2_ref):
  d = lax.rsqrt(c0_ref[...] + c1_ref[...] + 1.0)
  t = d * jnp.dot(h_ref[...], w_ref[...], preferred_element_type=jnp.float32)
  u2_ref[0] = t[:, :LANES]
  u2_ref[1] = t[:, LANES:]


def _tc_u2(H1, c0, c1, W2):
  return pl.pallas_call(
      _u2_body,
      grid=(N // BM,),
      in_specs=[
          pl.BlockSpec((BM, 512), lambda i: (i, 0)),
          pl.BlockSpec((BM, 1), lambda i: (i, 0)),
          pl.BlockSpec((BM, 1), lambda i: (i, 0)),
          pl.BlockSpec((512, 256), lambda i: (0, 0)),
      ],
      out_specs=pl.BlockSpec((2, BM, LANES), lambda i: (0, i, 0)),
      out_shape=jax.ShapeDtypeStruct((2, N, LANES), jnp.float32),
  )(H1, c0, c1, W2)


def _final_body(a_ref, u2_ref, c0_ref, c1_ref, b_ref, z_ref):
  d = lax.rsqrt(c0_ref[...] + c1_ref[...] + 1.0)
  agg = jnp.concatenate([a_ref[0], a_ref[1]], axis=1)
  uu = jnp.concatenate([u2_ref[0], u2_ref[1]], axis=1)
  z_ref[...] = d * (agg + uu) + b_ref[...]


def _tc_final(aggp, u2, c0, c1, b2):
  return pl.pallas_call(
      _final_body,
      grid=(N // BM,),
      in_specs=[
          pl.BlockSpec((2, BM, LANES), lambda i: (0, i, 0)),
          pl.BlockSpec((2, BM, LANES), lambda i: (0, i, 0)),
          pl.BlockSpec((BM, 1), lambda i: (i, 0)),
          pl.BlockSpec((BM, 1), lambda i: (i, 0)),
          pl.BlockSpec((1, 256), lambda i: (0, 0)),
      ],
      out_specs=pl.BlockSpec((BM, 256), lambda i: (i, 0)),
      out_shape=jax.ShapeDtypeStruct((N, 256), jnp.float32),
  )(aggp, u2, c0, c1, b2)


@jax.jit
def kernel(x, edge_index, W1, b1, W2, b2):
  src = edge_index[0].astype(jnp.int32)
  dst = edge_index[1].astype(jnp.int32)
  src2 = src.reshape(NS, EPT)
  dst2 = dst.reshape(NS, EPT)
  dst2w = dst.reshape(NW, EPW)

  zrow = jnp.zeros((RPT, LANES), jnp.float32)
  zcnt = jnp.zeros((RPT, CW), jnp.float32)
  ones = jnp.ones((CCH, CW), jnp.float32)

  cnt = _sc_cnt(dst2w, zcnt, ones)
  c0 = cnt[0, :N, :1]
  c1 = cnt[1, :N, :1]

  u = _tc_prep(x, c0, c1)
  agg1 = _sc_agg(u, src2, dst2, zrow)
  H1 = _tc_layer1(agg1, u, c0, c1, W1, b1.reshape(1, 512))
  u2 = _tc_u2(H1, c0, c1, W2)
  agg2 = _sc_agg(u2, src2, dst2, zrow)
  z = _tc_final(agg2, u2, c0, c1, b2.reshape(1, 256))
  return z


# speedup vs baseline: 1.0532x; 1.0532x over previous
import jax
import jax.numpy as jnp
from jax import lax
from jax.experimental import pallas as pl
from jax.experimental.pallas import tpu as pltpu
from jax.experimental.pallas import tpu_sc as plsc

N = 10000
E = 160000
NC = 2
NS = 16
LANES = 128

EPT = E // NS
CH = 80
NCHUNK = EPT // CH
NBUF = 2
NP = 10112
RPT = NP // NS

NW = NC * NS
EPW = E // NW
CW = 128
CCH = 40
NCC = EPW // CCH

_MESH = plsc.VectorSubcoreMesh(core_axis_name="c", subcore_axis_name="s")


def _cnt_body(dst2w_hbm, zcnt, ones_hbm, cnt_hbm, cntacc, dstlall, onesbuf):
  c = lax.axis_index("c")
  s = lax.axis_index("s")
  w = s * NC + c

  pltpu.sync_copy(zcnt, cntacc.at[pl.ds(s * RPT, RPT)])
  pltpu.sync_copy(ones_hbm, onesbuf)
  pltpu.sync_copy(dst2w_hbm.at[w], dstlall)
  plsc.subcore_barrier()

  def chunk(k, _):
    pltpu.sync_copy(onesbuf, cntacc.at[dstlall.at[pl.ds(k * CCH, CCH)]],
                    add=True)
    return 0
  lax.fori_loop(0, NCC, chunk, 0)

  plsc.subcore_barrier()
  pltpu.sync_copy(cntacc.at[pl.ds(s * RPT, RPT)],
                  cnt_hbm.at[c, pl.ds(s * RPT, RPT)])


_sc_cnt = pl.kernel(
    _cnt_body,
    out_type=jax.ShapeDtypeStruct((NC, NP, CW), jnp.float32),
    mesh=_MESH,
    scratch_types=[
        pltpu.VMEM_SHARED((NP, CW), jnp.float32),
        pltpu.VMEM((EPW,), jnp.int32),
        pltpu.VMEM((CCH, CW), jnp.float32),
    ],
)


def _agg_body(u3_hbm, src2_hbm, dst2_hbm, zrow, agg_hbm, acc, srcall, dstall,
              b0, b1, s0, s1):
  c = lax.axis_index("c")
  s = lax.axis_index("s")
  bufs = (b0, b1)
  sems = (s0, s1)

  pltpu.sync_copy(zrow, acc.at[pl.ds(s * RPT, RPT)])
  pltpu.sync_copy(src2_hbm.at[s], srcall)
  pltpu.sync_copy(dst2_hbm.at[s], dstall)
  plsc.subcore_barrier()

  uc = u3_hbm.at[c]

  def fire(k, b):
    pltpu.async_copy(uc.at[srcall.at[pl.ds(k * CH, CH)]], bufs[b], sems[b])

  def finish(k, b):
    pltpu.make_async_copy(uc.at[srcall.at[pl.ds(0, CH)]],
                          bufs[b], sems[b]).wait()
    pltpu.sync_copy(bufs[b], acc.at[dstall.at[pl.ds(k * CH, CH)]],
                    add=True)

  fire(0, 0)
  fire(1, 1)

  def outer(j, _):
    for b in range(NBUF):
      k = j * NBUF + b
      finish(k, b)
      fire(k + NBUF, b)
    return 0
  lax.fori_loop(0, (NCHUNK - 3) // NBUF, outer, 0)

  finish(NCHUNK - 3, 0)
  fire(NCHUNK - 1, 0)
  finish(NCHUNK - 2, 1)
  finish(NCHUNK - 1, 0)

  plsc.subcore_barrier()
  pltpu.sync_copy(acc.at[pl.ds(s * RPT, RPT)],
                  agg_hbm.at[c, pl.ds(s * RPT, RPT)])


_sc_agg = pl.kernel(
    _agg_body,
    out_type=jax.ShapeDtypeStruct((NC, NP, LANES), jnp.float32),
    mesh=_MESH,
    scratch_types=[
        pltpu.VMEM_SHARED((NP, LANES), jnp.float32),
        pltpu.VMEM((EPT,), jnp.int32),
        pltpu.VMEM((EPT,), jnp.int32),
    ] + [pltpu.VMEM((CH, LANES), jnp.float32) for _ in range(NBUF)]
      + [pltpu.SemaphoreType.DMA for _ in range(NBUF)],
)


BM = 1000


def _prep_body(x_ref, c0_ref, c1_ref, u_ref, d_ref):
  d = lax.rsqrt(c0_ref[...] + c1_ref[...] + 1.0)
  ux = d * x_ref[...]
  u_ref[0] = ux[:, :LANES]
  u_ref[1] = ux[:, LANES:]
  d_ref[...] = d


def _tc_prep(x, c0, c1):
  return pl.pallas_call(
      _prep_body,
      grid=(N // BM,),
      in_specs=[
          pl.BlockSpec((BM, 256), lambda i: (i, 0)),
          pl.BlockSpec((BM, 1), lambda i: (i, 0)),
          pl.BlockSpec((BM, 1), lambda i: (i, 0)),
      ],
      out_specs=[
          pl.BlockSpec((2, BM, LANES), lambda i: (0, i, 0)),
          pl.BlockSpec((BM, 1), lambda i: (i, 0)),
      ],
      out_shape=[
          jax.ShapeDtypeStruct((2, N, LANES), jnp.float32),
          jax.ShapeDtypeStruct((N, 1), jnp.float32),
      ],
  )(x, c0, c1)


def _mid_body(a_ref, u_ref, d_ref, w1_ref, b1_ref, w2_ref, u2_ref):
  agg = jnp.concatenate([a_ref[0], a_ref[1]], axis=1)
  uu = jnp.concatenate([u_ref[0], u_ref[1]], axis=1)
  p = d_ref[...] * (agg + uu)
  h = jnp.dot(p, w1_ref[...], preferred_element_type=jnp.float32) + b1_ref[...]
  h = jnp.maximum(h, 0.0)
  t = d_ref[...] * jnp.dot(h, w2_ref[...], preferred_element_type=jnp.float32)
  u2_ref[0] = t[:, :LANES]
  u2_ref[1] = t[:, LANES:]


def _tc_mid(aggp, u, dcol, W1, b1, W2):
  return pl.pallas_call(
      _mid_body,
      grid=(N // BM,),
      in_specs=[
          pl.BlockSpec((2, BM, LANES), lambda i: (0, i, 0)),
          pl.BlockSpec((2, BM, LANES), lambda i: (0, i, 0)),
          pl.BlockSpec((BM, 1), lambda i: (i, 0)),
          pl.BlockSpec((256, 512), lambda i: (0, 0)),
          pl.BlockSpec((1, 512), lambda i: (0, 0)),
          pl.BlockSpec((512, 256), lambda i: (0, 0)),
      ],
      out_specs=pl.BlockSpec((2, BM, LANES), lambda i: (0, i, 0)),
      out_shape=jax.ShapeDtypeStruct((2, N, LANES), jnp.float32),
  )(aggp, u, dcol, W1, b1, W2)


def _final_body(a_ref, u2_ref, d_ref, b_ref, z_ref):
  d = d_ref[...]
  agg = jnp.concatenate([a_ref[0], a_ref[1]], axis=1)
  uu = jnp.concatenate([u2_ref[0], u2_ref[1]], axis=1)
  z_ref[...] = d * (agg + uu) + b_ref[...]


def _tc_final(aggp, u2, dcol, b2):
  return pl.pallas_call(
      _final_body,
      grid=(N // BM,),
      in_specs=[
          pl.BlockSpec((2, BM, LANES), lambda i: (0, i, 0)),
          pl.BlockSpec((2, BM, LANES), lambda i: (0, i, 0)),
          pl.BlockSpec((BM, 1), lambda i: (i, 0)),
          pl.BlockSpec((1, 256), lambda i: (0, 0)),
      ],
      out_specs=pl.BlockSpec((BM, 256), lambda i: (i, 0)),
      out_shape=jax.ShapeDtypeStruct((N, 256), jnp.float32),
  )(aggp, u2, dcol, b2)


@jax.jit
def kernel(x, edge_index, W1, b1, W2, b2):
  src = edge_index[0].astype(jnp.int32)
  dst = edge_index[1].astype(jnp.int32)
  src2 = src.reshape(NS, EPT)
  dst2 = dst.reshape(NS, EPT)
  dst2w = dst.reshape(NW, EPW)

  zrow = jnp.zeros((RPT, LANES), jnp.float32)
  zcnt = jnp.zeros((RPT, CW), jnp.float32)
  ones = jnp.ones((CCH, CW), jnp.float32)

  cnt = _sc_cnt(dst2w, zcnt, ones)
  c0 = cnt[0, :N, :1]
  c1 = cnt[1, :N, :1]

  u, dcol = _tc_prep(x, c0, c1)
  agg1 = _sc_agg(u, src2, dst2, zrow)
  u2 = _tc_mid(agg1, u, dcol, W1, b1.reshape(1, 512), W2)
  agg2 = _sc_agg(u2, src2, dst2, zrow)
  z = _tc_final(agg2, u2, dcol, b2.reshape(1, 256))
  return z


# speedup vs baseline: 1.2240x; 1.1622x over previous
import jax
import jax.numpy as jnp
from jax import lax
from jax.experimental import pallas as pl
from jax.experimental.pallas import tpu as pltpu
from jax.experimental.pallas import tpu_sc as plsc

N = 10000
E = 160000
NC = 2
NS = 16
LANES = 128

EPT = E // NS
CH = 40
NCHUNK = EPT // CH
NBUF = 5
NP = 10112
RPT = NP // NS

NW = NC * NS
EPW = E // NW
CW = 128
CCH = 40
NCC = EPW // CCH

_MESH = plsc.VectorSubcoreMesh(core_axis_name="c", subcore_axis_name="s")


def _cnt_body(dst2w_hbm, zcnt, ones_hbm, cnt_hbm, cntacc, dstlall, onesbuf):
  c = lax.axis_index("c")
  s = lax.axis_index("s")
  w = s * NC + c

  pltpu.sync_copy(zcnt, cntacc.at[pl.ds(s * RPT, RPT)])
  pltpu.sync_copy(ones_hbm, onesbuf)
  pltpu.sync_copy(dst2w_hbm.at[w], dstlall)
  plsc.subcore_barrier()

  def chunk(k, _):
    pltpu.sync_copy(onesbuf, cntacc.at[dstlall.at[pl.ds(k * CCH, CCH)]],
                    add=True)
    return 0
  lax.fori_loop(0, NCC, chunk, 0)

  plsc.subcore_barrier()
  pltpu.sync_copy(cntacc.at[pl.ds(s * RPT, RPT)],
                  cnt_hbm.at[c, pl.ds(s * RPT, RPT)])


_sc_cnt = pl.kernel(
    _cnt_body,
    out_type=jax.ShapeDtypeStruct((NC, NP, CW), jnp.float32),
    mesh=_MESH,
    scratch_types=[
        pltpu.VMEM_SHARED((NP, CW), jnp.float32),
        pltpu.VMEM((EPW,), jnp.int32),
        pltpu.VMEM((CCH, CW), jnp.float32),
    ],
)


def _agg_body(u3_hbm, src2_hbm, dst2_hbm, zrow, agg_hbm, acc, srcall, dstall,
              *bs):
  c = lax.axis_index("c")
  s = lax.axis_index("s")
  bufs = bs[:NBUF]
  sems = bs[NBUF:]

  pltpu.sync_copy(zrow, acc.at[pl.ds(s * RPT, RPT)])
  pltpu.sync_copy(src2_hbm.at[s], srcall)
  pltpu.sync_copy(dst2_hbm.at[s], dstall)
  plsc.subcore_barrier()

  uc = u3_hbm.at[c]

  def fire(k, b):
    pltpu.async_copy(uc.at[srcall.at[pl.ds(k * CH, CH)]], bufs[b], sems[b])

  def finish(k, b):
    pltpu.make_async_copy(uc.at[srcall.at[pl.ds(0, CH)]],
                          bufs[b], sems[b]).wait()
    pltpu.sync_copy(bufs[b], acc.at[dstall.at[pl.ds(k * CH, CH)]],
                    add=True)

  for b in range(NBUF):
    fire(b, b)

  def outer(j, _):
    for b in range(NBUF):
      k = j * NBUF + b
      finish(k, b)
      fire(k + NBUF, b)
    return 0
  lax.fori_loop(0, NCHUNK // NBUF - 2, outer, 0)

  for b in range(NBUF):
    k = NCHUNK - 2 * NBUF + b
    finish(k, b)
    fire(k + NBUF, b)
  for b in range(NBUF):
    finish(NCHUNK - NBUF + b, b)

  plsc.subcore_barrier()
  pltpu.sync_copy(acc.at[pl.ds(s * RPT, RPT)],
                  agg_hbm.at[c, pl.ds(s * RPT, RPT)])


_sc_agg = pl.kernel(
    _agg_body,
    out_type=jax.ShapeDtypeStruct((NC, NP, LANES), jnp.float32),
    mesh=_MESH,
    scratch_types=[
        pltpu.VMEM_SHARED((NP, LANES), jnp.float32),
        pltpu.VMEM((EPT,), jnp.int32),
        pltpu.VMEM((EPT,), jnp.int32),
    ] + [pltpu.VMEM((CH, LANES), jnp.float32) for _ in range(NBUF)]
      + [pltpu.SemaphoreType.DMA for _ in range(NBUF)],
)


BM = 1000


def _prep_body(x_ref, c0_ref, c1_ref, u_ref, d_ref):
  d = lax.rsqrt(c0_ref[...] + c1_ref[...] + 1.0)
  ux = d * x_ref[...]
  u_ref[0] = ux[:, :LANES]
  u_ref[1] = ux[:, LANES:]
  d_ref[...] = d


def _tc_prep(x, c0, c1):
  return pl.pallas_call(
      _prep_body,
      grid=(N // BM,),
      in_specs=[
          pl.BlockSpec((BM, 256), lambda i: (i, 0)),
          pl.BlockSpec((BM, 1), lambda i: (i, 0)),
          pl.BlockSpec((BM, 1), lambda i: (i, 0)),
      ],
      out_specs=[
          pl.BlockSpec((2, BM, LANES), lambda i: (0, i, 0)),
          pl.BlockSpec((BM, 1), lambda i: (i, 0)),
      ],
      out_shape=[
          jax.ShapeDtypeStruct((2, N, LANES), jnp.float32),
          jax.ShapeDtypeStruct((N, 1), jnp.float32),
      ],
  )(x, c0, c1)


def _mid_body(a_ref, u_ref, d_ref, w1_ref, b1_ref, w2_ref, u2_ref):
  agg = jnp.concatenate([a_ref[0], a_ref[1]], axis=1)
  uu = jnp.concatenate([u_ref[0], u_ref[1]], axis=1)
  p = d_ref[...] * (agg + uu)
  h = jnp.dot(p, w1_ref[...], preferred_element_type=jnp.float32) + b1_ref[...]
  h = jnp.maximum(h, 0.0)
  t = d_ref[...] * jnp.dot(h, w2_ref[...], preferred_element_type=jnp.float32)
  u2_ref[0] = t[:, :LANES]
  u2_ref[1] = t[:, LANES:]


def _tc_mid(aggp, u, dcol, W1, b1, W2):
  return pl.pallas_call(
      _mid_body,
      grid=(N // BM,),
      in_specs=[
          pl.BlockSpec((2, BM, LANES), lambda i: (0, i, 0)),
          pl.BlockSpec((2, BM, LANES), lambda i: (0, i, 0)),
          pl.BlockSpec((BM, 1), lambda i: (i, 0)),
          pl.BlockSpec((256, 512), lambda i: (0, 0)),
          pl.BlockSpec((1, 512), lambda i: (0, 0)),
          pl.BlockSpec((512, 256), lambda i: (0, 0)),
      ],
      out_specs=pl.BlockSpec((2, BM, LANES), lambda i: (0, i, 0)),
      out_shape=jax.ShapeDtypeStruct((2, N, LANES), jnp.float32),
  )(aggp, u, dcol, W1, b1, W2)


def _final_body(a_ref, u2_ref, d_ref, b_ref, z_ref):
  d = d_ref[...]
  agg = jnp.concatenate([a_ref[0], a_ref[1]], axis=1)
  uu = jnp.concatenate([u2_ref[0], u2_ref[1]], axis=1)
  z_ref[...] = d * (agg + uu) + b_ref[...]


def _tc_final(aggp, u2, dcol, b2):
  return pl.pallas_call(
      _final_body,
      grid=(N // BM,),
      in_specs=[
          pl.BlockSpec((2, BM, LANES), lambda i: (0, i, 0)),
          pl.BlockSpec((2, BM, LANES), lambda i: (0, i, 0)),
          pl.BlockSpec((BM, 1), lambda i: (i, 0)),
          pl.BlockSpec((1, 256), lambda i: (0, 0)),
      ],
      out_specs=pl.BlockSpec((BM, 256), lambda i: (i, 0)),
      out_shape=jax.ShapeDtypeStruct((N, 256), jnp.float32),
  )(aggp, u2, dcol, b2)


@jax.jit
def kernel(x, edge_index, W1, b1, W2, b2):
  src = edge_index[0].astype(jnp.int32)
  dst = edge_index[1].astype(jnp.int32)
  src2 = src.reshape(NS, EPT)
  dst2 = dst.reshape(NS, EPT)
  dst2w = dst.reshape(NW, EPW)

  zrow = jnp.zeros((RPT, LANES), jnp.float32)
  zcnt = jnp.zeros((RPT, CW), jnp.float32)
  ones = jnp.ones((CCH, CW), jnp.float32)

  cnt = _sc_cnt(dst2w, zcnt, ones)
  c0 = cnt[0, :N, :1]
  c1 = cnt[1, :N, :1]

  u, dcol = _tc_prep(x, c0, c1)
  agg1 = _sc_agg(u, src2, dst2, zrow)
  u2 = _tc_mid(agg1, u, dcol, W1, b1.reshape(1, 512), W2)
  agg2 = _sc_agg(u2, src2, dst2, zrow)
  z = _tc_final(agg2, u2, dcol, b2.reshape(1, 256))
  return z


# speedup vs baseline: 1.2457x; 1.0177x over previous
import jax
import jax.numpy as jnp
from jax import lax
from jax.experimental import pallas as pl
from jax.experimental.pallas import tpu as pltpu
from jax.experimental.pallas import tpu_sc as plsc

N = 10000
E = 160000
NC = 2
NS = 16
LANES = 128

EPT = E // NS
CH = 40
NCHUNK = EPT // CH
NBUF = 5
NP = 10112
RPT = NP // NS

NW = NC * NS
EPW = E // NW
CW = 128
CCH = 40
NCC = EPW // CCH

_MESH = plsc.VectorSubcoreMesh(core_axis_name="c", subcore_axis_name="s")


def _cnt_body(dst2w_hbm, zcnt, ones_hbm, cnt_hbm, cntacc, dstlall, onesbuf):
  c = lax.axis_index("c")
  s = lax.axis_index("s")
  w = s * NC + c

  pltpu.sync_copy(zcnt, cntacc.at[pl.ds(s * RPT, RPT)])
  pltpu.sync_copy(ones_hbm, onesbuf)
  pltpu.sync_copy(dst2w_hbm.at[w], dstlall)
  plsc.subcore_barrier()

  def chunk(k, _):
    pltpu.sync_copy(onesbuf, cntacc.at[dstlall.at[pl.ds(k * CCH, CCH)]],
                    add=True)
    return 0
  lax.fori_loop(0, NCC, chunk, 0)

  plsc.subcore_barrier()
  pltpu.sync_copy(cntacc.at[pl.ds(s * RPT, RPT)],
                  cnt_hbm.at[c, pl.ds(s * RPT, RPT)])


_sc_cnt = pl.kernel(
    _cnt_body,
    out_type=jax.ShapeDtypeStruct((NC, NP, CW), jnp.float32),
    mesh=_MESH,
    scratch_types=[
        pltpu.VMEM_SHARED((NP, CW), jnp.float32),
        pltpu.VMEM((EPW,), jnp.int32),
        pltpu.VMEM((CCH, CW), jnp.float32),
    ],
)


def _agg_body(u3_hbm, src2_hbm, dst2_hbm, zrow, agg_hbm, acc, srcall, dstall,
              *bs):
  c = lax.axis_index("c")
  s = lax.axis_index("s")
  bufs = bs[:NBUF]
  sems = bs[NBUF:]

  pltpu.sync_copy(zrow, acc.at[pl.ds(s * RPT, RPT)])
  pltpu.sync_copy(src2_hbm.at[s], srcall)
  pltpu.sync_copy(dst2_hbm.at[s], dstall)
  plsc.subcore_barrier()

  uc = u3_hbm.at[c]

  def fire(k, b):
    pltpu.async_copy(uc.at[srcall.at[pl.ds(k * CH, CH)]], bufs[b], sems[b])

  def finish(k, b):
    pltpu.make_async_copy(uc.at[srcall.at[pl.ds(0, CH)]],
                          bufs[b], sems[b]).wait()
    pltpu.sync_copy(bufs[b], acc.at[dstall.at[pl.ds(k * CH, CH)]],
                    add=True)

  for b in range(NBUF):
    fire(b, b)

  def outer(j, _):
    for b in range(NBUF):
      k = j * NBUF + b
      finish(k, b)
      fire(k + NBUF, b)
    return 0
  lax.fori_loop(0, NCHUNK // NBUF - 2, outer, 0)

  for b in range(NBUF):
    k = NCHUNK - 2 * NBUF + b
    finish(k, b)
    fire(k + NBUF, b)
  for b in range(NBUF):
    finish(NCHUNK - NBUF + b, b)

  plsc.subcore_barrier()
  pltpu.sync_copy(acc.at[pl.ds(s * RPT, RPT)],
                  agg_hbm.at[c, pl.ds(s * RPT, RPT)])


_sc_agg = pl.kernel(
    _agg_body,
    out_type=jax.ShapeDtypeStruct((NC, NP, LANES), jnp.float32),
    mesh=_MESH,
    scratch_types=[
        pltpu.VMEM_SHARED((NP, LANES), jnp.float32),
        pltpu.VMEM((EPT,), jnp.int32),
        pltpu.VMEM((EPT,), jnp.int32),
    ] + [pltpu.VMEM((CH, LANES), jnp.float32) for _ in range(NBUF)]
      + [pltpu.SemaphoreType.DMA for _ in range(NBUF)],
)


BM = 1000


def _prep_body(x_ref, c0_ref, c1_ref, u_ref, d_ref):
  d = lax.rsqrt(c0_ref[0, :, :1] + c1_ref[0, :, :1] + 1.0)
  ux = d * x_ref[...]
  u_ref[0] = ux[:, :LANES]
  u_ref[1] = ux[:, LANES:]
  d_ref[...] = d


def _tc_prep(x, c0, c1):
  return pl.pallas_call(
      _prep_body,
      grid=(N // BM,),
      in_specs=[
          pl.BlockSpec((BM, 256), lambda i: (i, 0)),
          pl.BlockSpec((1, BM, LANES), lambda i: (0, i, 0)),
          pl.BlockSpec((1, BM, LANES), lambda i: (1, i, 0)),
      ],
      out_specs=[
          pl.BlockSpec((2, BM, LANES), lambda i: (0, i, 0)),
          pl.BlockSpec((BM, 1), lambda i: (i, 0)),
      ],
      out_shape=[
          jax.ShapeDtypeStruct((2, N, LANES), jnp.float32),
          jax.ShapeDtypeStruct((N, 1), jnp.float32),
      ],
  )(x, c0, c1)


def _mid_body(a_ref, u_ref, d_ref, w1_ref, b1_ref, w2_ref, u2_ref):
  agg = jnp.concatenate([a_ref[0], a_ref[1]], axis=1)
  uu = jnp.concatenate([u_ref[0], u_ref[1]], axis=1)
  p = d_ref[...] * (agg + uu)
  h = jnp.dot(p, w1_ref[...], preferred_element_type=jnp.float32) + b1_ref[...]
  h = jnp.maximum(h, 0.0)
  t = d_ref[...] * jnp.dot(h, w2_ref[...], preferred_element_type=jnp.float32)
  u2_ref[0] = t[:, :LANES]
  u2_ref[1] = t[:, LANES:]


def _tc_mid(aggp, u, dcol, W1, b1, W2):
  return pl.pallas_call(
      _mid_body,
      grid=(N // BM,),
      in_specs=[
          pl.BlockSpec((2, BM, LANES), lambda i: (0, i, 0)),
          pl.BlockSpec((2, BM, LANES), lambda i: (0, i, 0)),
          pl.BlockSpec((BM, 1), lambda i: (i, 0)),
          pl.BlockSpec((256, 512), lambda i: (0, 0)),
          pl.BlockSpec((1, 512), lambda i: (0, 0)),
          pl.BlockSpec((512, 256), lambda i: (0, 0)),
      ],
      out_specs=pl.BlockSpec((2, BM, LANES), lambda i: (0, i, 0)),
      out_shape=jax.ShapeDtypeStruct((2, N, LANES), jnp.float32),
  )(aggp, u, dcol, W1, b1, W2)


def _final_body(a_ref, u2_ref, d_ref, b_ref, z_ref):
  d = d_ref[...]
  agg = jnp.concatenate([a_ref[0], a_ref[1]], axis=1)
  uu = jnp.concatenate([u2_ref[0], u2_ref[1]], axis=1)
  z_ref[...] = d * (agg + uu) + b_ref[...]


def _tc_final(aggp, u2, dcol, b2):
  return pl.pallas_call(
      _final_body,
      grid=(N // BM,),
      in_specs=[
          pl.BlockSpec((2, BM, LANES), lambda i: (0, i, 0)),
          pl.BlockSpec((2, BM, LANES), lambda i: (0, i, 0)),
          pl.BlockSpec((BM, 1), lambda i: (i, 0)),
          pl.BlockSpec((1, 256), lambda i: (0, 0)),
      ],
      out_specs=pl.BlockSpec((BM, 256), lambda i: (i, 0)),
      out_shape=jax.ShapeDtypeStruct((N, 256), jnp.float32),
  )(aggp, u2, dcol, b2)


@jax.jit
def kernel(x, edge_index, W1, b1, W2, b2):
  src = edge_index[0].astype(jnp.int32)
  dst = edge_index[1].astype(jnp.int32)
  src2 = src.reshape(NS, EPT)
  dst2 = dst.reshape(NS, EPT)
  dst2w = dst.reshape(NW, EPW)

  zrow = jnp.zeros((RPT, LANES), jnp.float32)
  zcnt = jnp.zeros((RPT, CW), jnp.float32)
  ones = jnp.ones((CCH, CW), jnp.float32)

  cnt = _sc_cnt(dst2w, zcnt, ones)
  u, dcol = _tc_prep(x, cnt, cnt)
  agg1 = _sc_agg(u, src2, dst2, zrow)
  u2 = _tc_mid(agg1, u, dcol, W1, b1.reshape(1, 512), W2)
  agg2 = _sc_agg(u2, src2, dst2, zrow)
  z = _tc_final(agg2, u2, dcol, b2.reshape(1, 256))
  return z


# speedup vs baseline: 1.2586x; 1.0103x over previous
import jax
import jax.numpy as jnp
from jax import lax
from jax.experimental import pallas as pl
from jax.experimental.pallas import tpu as pltpu
from jax.experimental.pallas import tpu_sc as plsc

N = 10000
E = 160000
NC = 2
NS = 16
LANES = 128

EPT = E // NS
CH = 40
NCHUNK = EPT // CH
NBUF = 5
NP = 10112
RPT = NP // NS

NW = NC * NS
EPW = E // NW
CW = 128
CCH = 40
NCC = EPW // CCH

_MESH = plsc.VectorSubcoreMesh(core_axis_name="c", subcore_axis_name="s")


def _cnt_body(dst2w_hbm, zcnt, ones_hbm, cnt_hbm, cntacc, dstlall, onesbuf):
  c = lax.axis_index("c")
  s = lax.axis_index("s")
  w = s * NC + c

  pltpu.sync_copy(zcnt, cntacc.at[pl.ds(s * RPT, RPT)])
  pltpu.sync_copy(ones_hbm, onesbuf)
  pltpu.sync_copy(dst2w_hbm.at[w], dstlall)
  plsc.subcore_barrier()

  def chunk(k, _):
    pltpu.sync_copy(onesbuf, cntacc.at[dstlall.at[pl.ds(k * CCH, CCH)]],
                    add=True)
    return 0
  lax.fori_loop(0, NCC, chunk, 0)

  plsc.subcore_barrier()
  pltpu.sync_copy(cntacc.at[pl.ds(s * RPT, RPT)],
                  cnt_hbm.at[c, pl.ds(s * RPT, RPT)])


_sc_cnt = pl.kernel(
    _cnt_body,
    out_type=jax.ShapeDtypeStruct((NC, NP, CW), jnp.float32),
    mesh=_MESH,
    scratch_types=[
        pltpu.VMEM_SHARED((NP, CW), jnp.float32),
        pltpu.VMEM((EPW,), jnp.int32),
        pltpu.VMEM((CCH, CW), jnp.float32),
    ],
)


def _agg_body(u3_hbm, src2_hbm, dst2_hbm, zrow, agg_hbm, acc, srcall, dstall,
              *bs):
  c = lax.axis_index("c")
  s = lax.axis_index("s")
  bufs = bs[:NBUF]
  sems = bs[NBUF + 3:]

  z = pltpu.async_copy(zrow, acc.at[pl.ds(s * RPT, RPT)], bs[NBUF])
  a = pltpu.async_copy(src2_hbm.at[s], srcall, bs[NBUF + 1])
  b = pltpu.async_copy(dst2_hbm.at[s], dstall, bs[NBUF + 2])
  z.wait()
  a.wait()
  b.wait()
  plsc.subcore_barrier()

  uc = u3_hbm.at[c]

  def fire(k, b):
    pltpu.async_copy(uc.at[srcall.at[pl.ds(k * CH, CH)]], bufs[b], sems[b])

  def finish(k, b):
    pltpu.make_async_copy(uc.at[srcall.at[pl.ds(0, CH)]],
                          bufs[b], sems[b]).wait()
    pltpu.sync_copy(bufs[b], acc.at[dstall.at[pl.ds(k * CH, CH)]],
                    add=True)

  for b in range(NBUF):
    fire(b, b)

  def outer(j, _):
    for b in range(NBUF):
      k = j * NBUF + b
      finish(k, b)
      fire(k + NBUF, b)
    return 0
  lax.fori_loop(0, NCHUNK // NBUF - 2, outer, 0)

  for b in range(NBUF):
    k = NCHUNK - 2 * NBUF + b
    finish(k, b)
    fire(k + NBUF, b)
  for b in range(NBUF):
    finish(NCHUNK - NBUF + b, b)

  plsc.subcore_barrier()
  pltpu.sync_copy(acc.at[pl.ds(s * RPT, RPT)],
                  agg_hbm.at[c, pl.ds(s * RPT, RPT)])


_sc_agg = pl.kernel(
    _agg_body,
    out_type=jax.ShapeDtypeStruct((NC, NP, LANES), jnp.float32),
    mesh=_MESH,
    scratch_types=[
        pltpu.VMEM_SHARED((NP, LANES), jnp.float32),
        pltpu.VMEM((EPT,), jnp.int32),
        pltpu.VMEM((EPT,), jnp.int32),
    ] + [pltpu.VMEM((CH, LANES), jnp.float32) for _ in range(NBUF)]
      + [pltpu.SemaphoreType.DMA for _ in range(NBUF + 3)],
)


BM = 1000


def _prep_body(x_ref, c0_ref, c1_ref, u_ref, d_ref):
  d = lax.rsqrt(c0_ref[0, :, :1] + c1_ref[0, :, :1] + 1.0)
  ux = d * x_ref[...]
  u_ref[0] = ux[:, :LANES]
  u_ref[1] = ux[:, LANES:]
  d_ref[...] = d


def _tc_prep(x, c0, c1):
  return pl.pallas_call(
      _prep_body,
      grid=(N // BM,),
      in_specs=[
          pl.BlockSpec((BM, 256), lambda i: (i, 0)),
          pl.BlockSpec((1, BM, LANES), lambda i: (0, i, 0)),
          pl.BlockSpec((1, BM, LANES), lambda i: (1, i, 0)),
      ],
      out_specs=[
          pl.BlockSpec((2, BM, LANES), lambda i: (0, i, 0)),
          pl.BlockSpec((BM, 1), lambda i: (i, 0)),
      ],
      out_shape=[
          jax.ShapeDtypeStruct((2, N, LANES), jnp.float32),
          jax.ShapeDtypeStruct((N, 1), jnp.float32),
      ],
  )(x, c0, c1)


def _mid_body(a_ref, u_ref, d_ref, w1_ref, b1_ref, w2_ref, u2_ref):
  agg = jnp.concatenate([a_ref[0], a_ref[1]], axis=1)
  uu = jnp.concatenate([u_ref[0], u_ref[1]], axis=1)
  p = d_ref[...] * (agg + uu)
  h = jnp.dot(p, w1_ref[...], preferred_element_type=jnp.float32) + b1_ref[...]
  h = jnp.maximum(h, 0.0)
  t = d_ref[...] * jnp.dot(h, w2_ref[...], preferred_element_type=jnp.float32)
  u2_ref[0] = t[:, :LANES]
  u2_ref[1] = t[:, LANES:]


def _tc_mid(aggp, u, dcol, W1, b1, W2):
  return pl.pallas_call(
      _mid_body,
      grid=(N // BM,),
      in_specs=[
          pl.BlockSpec((2, BM, LANES), lambda i: (0, i, 0)),
          pl.BlockSpec((2, BM, LANES), lambda i: (0, i, 0)),
          pl.BlockSpec((BM, 1), lambda i: (i, 0)),
          pl.BlockSpec((256, 512), lambda i: (0, 0)),
          pl.BlockSpec((1, 512), lambda i: (0, 0)),
          pl.BlockSpec((512, 256), lambda i: (0, 0)),
      ],
      out_specs=pl.BlockSpec((2, BM, LANES), lambda i: (0, i, 0)),
      out_shape=jax.ShapeDtypeStruct((2, N, LANES), jnp.float32),
  )(aggp, u, dcol, W1, b1, W2)


def _final_body(a_ref, u2_ref, d_ref, b_ref, z_ref):
  d = d_ref[...]
  agg = jnp.concatenate([a_ref[0], a_ref[1]], axis=1)
  uu = jnp.concatenate([u2_ref[0], u2_ref[1]], axis=1)
  z_ref[...] = d * (agg + uu) + b_ref[...]


def _tc_final(aggp, u2, dcol, b2):
  return pl.pallas_call(
      _final_body,
      grid=(N // BM,),
      in_specs=[
          pl.BlockSpec((2, BM, LANES), lambda i: (0, i, 0)),
          pl.BlockSpec((2, BM, LANES), lambda i: (0, i, 0)),
          pl.BlockSpec((BM, 1), lambda i: (i, 0)),
          pl.BlockSpec((1, 256), lambda i: (0, 0)),
      ],
      out_specs=pl.BlockSpec((BM, 256), lambda i: (i, 0)),
      out_shape=jax.ShapeDtypeStruct((N, 256), jnp.float32),
  )(aggp, u2, dcol, b2)


@jax.jit
def kernel(x, edge_index, W1, b1, W2, b2):
  src = edge_index[0].astype(jnp.int32)
  dst = edge_index[1].astype(jnp.int32)
  src2 = src.reshape(NS, EPT)
  dst2 = dst.reshape(NS, EPT)
  dst2w = dst.reshape(NW, EPW)

  zrow = jnp.zeros((RPT, LANES), jnp.float32)
  zcnt = jnp.zeros((RPT, CW), jnp.float32)
  ones = jnp.ones((CCH, CW), jnp.float32)

  cnt = _sc_cnt(dst2w, zcnt, ones)
  u, dcol = _tc_prep(x, cnt, cnt)
  agg1 = _sc_agg(u, src2, dst2, zrow)
  u2 = _tc_mid(agg1, u, dcol, W1, b1.reshape(1, 512), W2)
  agg2 = _sc_agg(u2, src2, dst2, zrow)
  z = _tc_final(agg2, u2, dcol, b2.reshape(1, 256))
  return z


# speedup vs baseline: 1.2667x; 1.0064x over previous
import jax
import jax.numpy as jnp
from jax import lax
from jax.experimental import pallas as pl
from jax.experimental.pallas import tpu as pltpu
from jax.experimental.pallas import tpu_sc as plsc

N = 10000
E = 160000
NC = 2
NS = 16
LANES = 128

EPT = E // NS
CH = 40
NCHUNK = EPT // CH
NBUF = 5
NP = 10112
RPT = NP // NS

NW = NC * NS
EPW = E // NW
CW = 128
CCH = 40
NCC = EPW // CCH

_MESH = plsc.VectorSubcoreMesh(core_axis_name="c", subcore_axis_name="s")


def _cnt_body(dst2w_hbm, zcnt, ones_hbm, cnt_hbm, cntacc, dstlall, onesbuf,
              sz, so, sd):
  c = lax.axis_index("c")
  s = lax.axis_index("s")
  w = s * NC + c

  a = pltpu.async_copy(zcnt, cntacc.at[pl.ds(s * RPT, RPT)], sz)
  b = pltpu.async_copy(ones_hbm, onesbuf, so)
  e = pltpu.async_copy(dst2w_hbm.at[w], dstlall, sd)
  a.wait()
  b.wait()
  e.wait()
  plsc.subcore_barrier()

  def chunk(k, _):
    pltpu.sync_copy(onesbuf, cntacc.at[dstlall.at[pl.ds(k * CCH, CCH)]],
                    add=True)
    return 0
  lax.fori_loop(0, NCC, chunk, 0)

  plsc.subcore_barrier()
  pltpu.sync_copy(cntacc.at[pl.ds(s * RPT, RPT)],
                  cnt_hbm.at[c, pl.ds(s * RPT, RPT)])


_sc_cnt = pl.kernel(
    _cnt_body,
    out_type=jax.ShapeDtypeStruct((NC, NP, CW), jnp.float32),
    mesh=_MESH,
    scratch_types=[
        pltpu.VMEM_SHARED((NP, CW), jnp.float32),
        pltpu.VMEM((EPW,), jnp.int32),
        pltpu.VMEM((CCH, CW), jnp.float32),
        pltpu.SemaphoreType.DMA,
        pltpu.SemaphoreType.DMA,
        pltpu.SemaphoreType.DMA,
    ],
)


def _agg_body(u3_hbm, src2_hbm, dst2_hbm, zrow, agg_hbm, acc, srcall, dstall,
              *bs):
  c = lax.axis_index("c")
  s = lax.axis_index("s")
  bufs = bs[:NBUF]
  sems = bs[NBUF + 3:]

  z = pltpu.async_copy(zrow, acc.at[pl.ds(s * RPT, RPT)], bs[NBUF])
  a = pltpu.async_copy(src2_hbm.at[s], srcall, bs[NBUF + 1])
  b = pltpu.async_copy(dst2_hbm.at[s], dstall, bs[NBUF + 2])
  z.wait()
  a.wait()
  b.wait()
  plsc.subcore_barrier()

  uc = u3_hbm.at[c]

  def fire(k, b):
    pltpu.async_copy(uc.at[srcall.at[pl.ds(k * CH, CH)]], bufs[b], sems[b])

  def finish(k, b):
    pltpu.make_async_copy(uc.at[srcall.at[pl.ds(0, CH)]],
                          bufs[b], sems[b]).wait()
    pltpu.sync_copy(bufs[b], acc.at[dstall.at[pl.ds(k * CH, CH)]],
                    add=True)

  for b in range(NBUF):
    fire(b, b)

  def outer(j, _):
    for b in range(NBUF):
      k = j * NBUF + b
      finish(k, b)
      fire(k + NBUF, b)
    return 0
  lax.fori_loop(0, NCHUNK // NBUF - 2, outer, 0)

  for b in range(NBUF):
    k = NCHUNK - 2 * NBUF + b
    finish(k, b)
    fire(k + NBUF, b)
  for b in range(NBUF):
    finish(NCHUNK - NBUF + b, b)

  plsc.subcore_barrier()
  pltpu.sync_copy(acc.at[pl.ds(s * RPT, RPT)],
                  agg_hbm.at[c, pl.ds(s * RPT, RPT)])


_sc_agg = pl.kernel(
    _agg_body,
    out_type=jax.ShapeDtypeStruct((NC, NP, LANES), jnp.float32),
    mesh=_MESH,
    scratch_types=[
        pltpu.VMEM_SHARED((NP, LANES), jnp.float32),
        pltpu.VMEM((EPT,), jnp.int32),
        pltpu.VMEM((EPT,), jnp.int32),
    ] + [pltpu.VMEM((CH, LANES), jnp.float32) for _ in range(NBUF)]
      + [pltpu.SemaphoreType.DMA for _ in range(NBUF + 3)],
)


BM = 1000


def _prep_body(x_ref, c0_ref, c1_ref, u_ref, d_ref):
  d = lax.rsqrt(c0_ref[0, :, :1] + c1_ref[0, :, :1] + 1.0)
  ux = d * x_ref[...]
  u_ref[0] = ux[:, :LANES]
  u_ref[1] = ux[:, LANES:]
  d_ref[...] = d


def _tc_prep(x, c0, c1):
  return pl.pallas_call(
      _prep_body,
      grid=(N // BM,),
      in_specs=[
          pl.BlockSpec((BM, 256), lambda i: (i, 0)),
          pl.BlockSpec((1, BM, LANES), lambda i: (0, i, 0)),
          pl.BlockSpec((1, BM, LANES), lambda i: (1, i, 0)),
      ],
      out_specs=[
          pl.BlockSpec((2, BM, LANES), lambda i: (0, i, 0)),
          pl.BlockSpec((BM, 1), lambda i: (i, 0)),
      ],
      out_shape=[
          jax.ShapeDtypeStruct((2, N, LANES), jnp.float32),
          jax.ShapeDtypeStruct((N, 1), jnp.float32),
      ],
  )(x, c0, c1)


def _mid_body(a_ref, u_ref, d_ref, w1_ref, b1_ref, w2_ref, u2_ref):
  agg = jnp.concatenate([a_ref[0], a_ref[1]], axis=1)
  uu = jnp.concatenate([u_ref[0], u_ref[1]], axis=1)
  p = d_ref[...] * (agg + uu)
  h = jnp.dot(p, w1_ref[...], preferred_element_type=jnp.float32) + b1_ref[...]
  h = jnp.maximum(h, 0.0)
  t = d_ref[...] * jnp.dot(h, w2_ref[...], preferred_element_type=jnp.float32)
  u2_ref[0] = t[:, :LANES]
  u2_ref[1] = t[:, LANES:]


def _tc_mid(aggp, u, dcol, W1, b1, W2):
  return pl.pallas_call(
      _mid_body,
      grid=(N // BM,),
      in_specs=[
          pl.BlockSpec((2, BM, LANES), lambda i: (0, i, 0)),
          pl.BlockSpec((2, BM, LANES), lambda i: (0, i, 0)),
          pl.BlockSpec((BM, 1), lambda i: (i, 0)),
          pl.BlockSpec((256, 512), lambda i: (0, 0)),
          pl.BlockSpec((1, 512), lambda i: (0, 0)),
          pl.BlockSpec((512, 256), lambda i: (0, 0)),
      ],
      out_specs=pl.BlockSpec((2, BM, LANES), lambda i: (0, i, 0)),
      out_shape=jax.ShapeDtypeStruct((2, N, LANES), jnp.float32),
  )(aggp, u, dcol, W1, b1, W2)


def _final_body(a_ref, u2_ref, d_ref, b_ref, z_ref):
  d = d_ref[...]
  agg = jnp.concatenate([a_ref[0], a_ref[1]], axis=1)
  uu = jnp.concatenate([u2_ref[0], u2_ref[1]], axis=1)
  z_ref[...] = d * (agg + uu) + b_ref[...]


def _tc_final(aggp, u2, dcol, b2):
  return pl.pallas_call(
      _final_body,
      grid=(N // BM,),
      in_specs=[
          pl.BlockSpec((2, BM, LANES), lambda i: (0, i, 0)),
          pl.BlockSpec((2, BM, LANES), lambda i: (0, i, 0)),
          pl.BlockSpec((BM, 1), lambda i: (i, 0)),
          pl.BlockSpec((1, 256), lambda i: (0, 0)),
      ],
      out_specs=pl.BlockSpec((BM, 256), lambda i: (i, 0)),
      out_shape=jax.ShapeDtypeStruct((N, 256), jnp.float32),
  )(aggp, u2, dcol, b2)


@jax.jit
def kernel(x, edge_index, W1, b1, W2, b2):
  src = edge_index[0].astype(jnp.int32)
  dst = edge_index[1].astype(jnp.int32)
  src2 = src.reshape(NS, EPT)
  dst2 = dst.reshape(NS, EPT)
  dst2w = dst.reshape(NW, EPW)

  zrow = jnp.zeros((RPT, LANES), jnp.float32)
  zcnt = jnp.zeros((RPT, CW), jnp.float32)
  ones = jnp.ones((CCH, CW), jnp.float32)

  cnt = _sc_cnt(dst2w, zcnt, ones)
  u, dcol = _tc_prep(x, cnt, cnt)
  agg1 = _sc_agg(u, src2, dst2, zrow)
  u2 = _tc_mid(agg1, u, dcol, W1, b1.reshape(1, 512), W2)
  agg2 = _sc_agg(u2, src2, dst2, zrow)
  z = _tc_final(agg2, u2, dcol, b2.reshape(1, 256))
  return z


# speedup vs baseline: 1.2847x; 1.0143x over previous
import jax
import jax.numpy as jnp
from jax import lax
from jax.experimental import pallas as pl
from jax.experimental.pallas import tpu as pltpu
from jax.experimental.pallas import tpu_sc as plsc

N = 10000
E = 160000
NC = 2
NS = 16
LANES = 128

EPT = E // NS
CH = 40
NCHUNK = EPT // CH
NBUF = 5
NP = 10112
RPT = NP // NS

NW = NC * NS
EPW = E // NW
CW = 128
CCH = 40
NCC = EPW // CCH

_MESH = plsc.VectorSubcoreMesh(core_axis_name="c", subcore_axis_name="s")


def _cnt_body(dst2w_hbm, zcnt, ones_hbm, cnt_hbm, cntacc, dstlall, onesbuf,
              sz, so, sd):
  c = lax.axis_index("c")
  s = lax.axis_index("s")
  w = s * NC + c

  a = pltpu.async_copy(zcnt, cntacc.at[pl.ds(s * RPT, RPT)], sz)
  b = pltpu.async_copy(ones_hbm, onesbuf, so)
  e = pltpu.async_copy(dst2w_hbm.at[w], dstlall, sd)
  a.wait()
  b.wait()
  e.wait()
  plsc.subcore_barrier()

  def group(g, _):
    descs = []
    for t in range(5):
      k = g * 5 + t
      descs.append(pltpu.async_copy(
          onesbuf, cntacc.at[dstlall.at[pl.ds(k * CCH, CCH)]], sz, add=True))
    for dsc in descs:
      dsc.wait()
    return 0
  lax.fori_loop(0, NCC // 5, group, 0)

  plsc.subcore_barrier()
  pltpu.sync_copy(cntacc.at[pl.ds(s * RPT, RPT)],
                  cnt_hbm.at[c, pl.ds(s * RPT, RPT)])


_sc_cnt = pl.kernel(
    _cnt_body,
    out_type=jax.ShapeDtypeStruct((NC, NP, CW), jnp.float32),
    mesh=_MESH,
    scratch_types=[
        pltpu.VMEM_SHARED((NP, CW), jnp.float32),
        pltpu.VMEM((EPW,), jnp.int32),
        pltpu.VMEM((CCH, CW), jnp.float32),
        pltpu.SemaphoreType.DMA,
        pltpu.SemaphoreType.DMA,
        pltpu.SemaphoreType.DMA,
    ],
)


def _agg_body(u3_hbm, src2_hbm, dst2_hbm, zrow, agg_hbm, acc, srcall, dstall,
              *bs):
  c = lax.axis_index("c")
  s = lax.axis_index("s")
  bufs = bs[:NBUF]
  sems = bs[NBUF + 3:]

  z = pltpu.async_copy(zrow, acc.at[pl.ds(s * RPT, RPT)], bs[NBUF])
  a = pltpu.async_copy(src2_hbm.at[s], srcall, bs[NBUF + 1])
  b = pltpu.async_copy(dst2_hbm.at[s], dstall, bs[NBUF + 2])
  z.wait()
  a.wait()
  b.wait()
  plsc.subcore_barrier()

  uc = u3_hbm.at[c]

  def fire(k, b):
    pltpu.async_copy(uc.at[srcall.at[pl.ds(k * CH, CH)]], bufs[b], sems[b])

  def finish(k, b):
    pltpu.make_async_copy(uc.at[srcall.at[pl.ds(0, CH)]],
                          bufs[b], sems[b]).wait()
    pltpu.sync_copy(bufs[b], acc.at[dstall.at[pl.ds(k * CH, CH)]],
                    add=True)

  for b in range(NBUF):
    fire(b, b)

  def outer(j, _):
    for b in range(NBUF):
      k = j * NBUF + b
      finish(k, b)
      fire(k + NBUF, b)
    return 0
  lax.fori_loop(0, NCHUNK // NBUF - 2, outer, 0)

  for b in range(NBUF):
    k = NCHUNK - 2 * NBUF + b
    finish(k, b)
    fire(k + NBUF, b)
  for b in range(NBUF):
    finish(NCHUNK - NBUF + b, b)

  plsc.subcore_barrier()
  pltpu.sync_copy(acc.at[pl.ds(s * RPT, RPT)],
                  agg_hbm.at[c, pl.ds(s * RPT, RPT)])


_sc_agg = pl.kernel(
    _agg_body,
    out_type=jax.ShapeDtypeStruct((NC, NP, LANES), jnp.float32),
    mesh=_MESH,
    scratch_types=[
        pltpu.VMEM_SHARED((NP, LANES), jnp.float32),
        pltpu.VMEM((EPT,), jnp.int32),
        pltpu.VMEM((EPT,), jnp.int32),
    ] + [pltpu.VMEM((CH, LANES), jnp.float32) for _ in range(NBUF)]
      + [pltpu.SemaphoreType.DMA for _ in range(NBUF + 3)],
)


BM = 1000


def _prep_body(x_ref, c0_ref, c1_ref, u_ref, d_ref):
  d = lax.rsqrt(c0_ref[0, :, :1] + c1_ref[0, :, :1] + 1.0)
  ux = d * x_ref[...]
  u_ref[0] = ux[:, :LANES]
  u_ref[1] = ux[:, LANES:]
  d_ref[...] = d


def _tc_prep(x, c0, c1):
  return pl.pallas_call(
      _prep_body,
      grid=(N // BM,),
      in_specs=[
          pl.BlockSpec((BM, 256), lambda i: (i, 0)),
          pl.BlockSpec((1, BM, LANES), lambda i: (0, i, 0)),
          pl.BlockSpec((1, BM, LANES), lambda i: (1, i, 0)),
      ],
      out_specs=[
          pl.BlockSpec((2, BM, LANES), lambda i: (0, i, 0)),
          pl.BlockSpec((BM, 1), lambda i: (i, 0)),
      ],
      out_shape=[
          jax.ShapeDtypeStruct((2, N, LANES), jnp.float32),
          jax.ShapeDtypeStruct((N, 1), jnp.float32),
      ],
  )(x, c0, c1)


def _mid_body(a_ref, u_ref, d_ref, w1_ref, b1_ref, w2_ref, u2_ref):
  agg = jnp.concatenate([a_ref[0], a_ref[1]], axis=1)
  uu = jnp.concatenate([u_ref[0], u_ref[1]], axis=1)
  p = d_ref[...] * (agg + uu)
  h = jnp.dot(p, w1_ref[...], preferred_element_type=jnp.float32) + b1_ref[...]
  h = jnp.maximum(h, 0.0)
  t = d_ref[...] * jnp.dot(h, w2_ref[...], preferred_element_type=jnp.float32)
  u2_ref[0] = t[:, :LANES]
  u2_ref[1] = t[:, LANES:]


def _tc_mid(aggp, u, dcol, W1, b1, W2):
  return pl.pallas_call(
      _mid_body,
      grid=(N // BM,),
      in_specs=[
          pl.BlockSpec((2, BM, LANES), lambda i: (0, i, 0)),
          pl.BlockSpec((2, BM, LANES), lambda i: (0, i, 0)),
          pl.BlockSpec((BM, 1), lambda i: (i, 0)),
          pl.BlockSpec((256, 512), lambda i: (0, 0)),
          pl.BlockSpec((1, 512), lambda i: (0, 0)),
          pl.BlockSpec((512, 256), lambda i: (0, 0)),
      ],
      out_specs=pl.BlockSpec((2, BM, LANES), lambda i: (0, i, 0)),
      out_shape=jax.ShapeDtypeStruct((2, N, LANES), jnp.float32),
  )(aggp, u, dcol, W1, b1, W2)


def _final_body(a_ref, u2_ref, d_ref, b_ref, z_ref):
  d = d_ref[...]
  agg = jnp.concatenate([a_ref[0], a_ref[1]], axis=1)
  uu = jnp.concatenate([u2_ref[0], u2_ref[1]], axis=1)
  z_ref[...] = d * (agg + uu) + b_ref[...]


def _tc_final(aggp, u2, dcol, b2):
  return pl.pallas_call(
      _final_body,
      grid=(N // BM,),
      in_specs=[
          pl.BlockSpec((2, BM, LANES), lambda i: (0, i, 0)),
          pl.BlockSpec((2, BM, LANES), lambda i: (0, i, 0)),
          pl.BlockSpec((BM, 1), lambda i: (i, 0)),
          pl.BlockSpec((1, 256), lambda i: (0, 0)),
      ],
      out_specs=pl.BlockSpec((BM, 256), lambda i: (i, 0)),
      out_shape=jax.ShapeDtypeStruct((N, 256), jnp.float32),
  )(aggp, u2, dcol, b2)


@jax.jit
def kernel(x, edge_index, W1, b1, W2, b2):
  src = edge_index[0].astype(jnp.int32)
  dst = edge_index[1].astype(jnp.int32)
  src2 = src.reshape(NS, EPT)
  dst2 = dst.reshape(NS, EPT)
  dst2w = dst.reshape(NW, EPW)

  zrow = jnp.zeros((RPT, LANES), jnp.float32)
  zcnt = jnp.zeros((RPT, CW), jnp.float32)
  ones = jnp.ones((CCH, CW), jnp.float32)

  cnt = _sc_cnt(dst2w, zcnt, ones)
  u, dcol = _tc_prep(x, cnt, cnt)
  agg1 = _sc_agg(u, src2, dst2, zrow)
  u2 = _tc_mid(agg1, u, dcol, W1, b1.reshape(1, 512), W2)
  agg2 = _sc_agg(u2, src2, dst2, zrow)
  z = _tc_final(agg2, u2, dcol, b2.reshape(1, 256))
  return z
